# baseline (speedup 1.0000x reference)
import functools
import math

import jax
import jax.numpy as jnp
from jax import lax
from jax.experimental import pallas as pl
from jax.experimental.pallas import tpu as pltpu
from jax.experimental.pallas import tpu_sc as plsc

DIM = 3
LVLS = 16
T = 524288
N_MIN = 16
N_MAX = 2048
F = 2


def _is_prime(n):
    if n < 2:
        return False
    if n % 2 == 0:
        return n == 2
    i = 3
    while i * i <= n:
        if n % i == 0:
            return False
        i += 2
    return True


def _next_prime(n):
    while not _is_prime(n):
        n += 1
    return n


P1 = _next_prime(1 << 17)
P2 = _next_prime(1 << 18)

_b = math.exp((math.log(N_MAX) - math.log(N_MIN)) / (LVLS - 1))
RES = []
METH = []
OFF = [0]
for _i in range(LVLS):
    _r = int(N_MIN * _b ** _i)
    RES.append(_r)
    _ne = (_r + 1) ** 2
    if _ne <= T:
        METH.append("one")
    else:
        METH.append("hash")
        _ne = T
    OFF.append(OFF[-1] + _ne)
ROWS = OFF[-1]
RMAX = ROWS - 1
MASK = T - 1

NPTS = 131072
NW = 32
PW = NPTS // NW
C = 64
NCH = PW // C
NG = C // 16
GB = 1024

N_STG_ONE = 6
PASSES = [
    (list(range(0, N_STG_ONE)), "spm"),
    ([12], "spm"),
    ([13], "spm"),
    ([14], "spm"),
    ([15], "spm"),
    (list(range(N_STG_ONE, 12)), "hbm"),
]
KMAX = max(len(p[0]) for p in PASSES)
NFLT_MAX = KMAX * 8 * 2 * C

BNC = 512
Q = 16 * BNC


def _padq(n):
    return -(-n // Q) * Q


STG = {}
_w = 0
for _l in range(N_STG_ONE):
    _start = RES[_l] ** 3 + OFF[_l]
    _size = RES[_l] ** 3 + RES[_l] ** 2 + RES[_l] + 1
    _start_a = _start - (_start % 8)
    _size_a = -(-(_size + (_start % 8)) // 8) * 8
    _fpad = _padq(2 * _size_a)
    STG[_l] = (_start_a, _fpad, _w)
    _w += _fpad
P0_FLOATS = _w
HSTG = {}
for _l in range(12, 16):
    _off_a = OFF[_l] - (OFF[_l] % 8)
    HSTG[_l] = (_off_a, _padq(2 * (T + 8)), OFF[_l] % 8)
SPMW = max(P0_FLOATS, _padq(2 * (T + 8)))
LATF_PAD = 2 * ROWS + Q + 64


@functools.cache
def _build_encoder():
    mesh = plsc.VectorSubcoreMesh(core_axis_name="c", subcore_axis_name="s")

    @functools.partial(
        pl.kernel,
        out_type=jax.ShapeDtypeStruct((NPTS * 2 * LVLS,), jnp.float32),
        mesh=mesh,
        scratch_types=[
            pltpu.VMEM_SHARED((SPMW,), jnp.float32),
            pltpu.VMEM((BNC,), jnp.float32),
            pltpu.VMEM((C,), jnp.float32),
            pltpu.VMEM((C,), jnp.float32),
            pltpu.VMEM((C,), jnp.float32),
            pltpu.VMEM((NFLT_MAX,), jnp.int32),
            pltpu.VMEM((NFLT_MAX // 2,), jnp.float32),
            pltpu.VMEM((NFLT_MAX,), jnp.float32),
            pltpu.VMEM((KMAX * 2 * C,), jnp.float32),
            pltpu.SemaphoreType.DMA,
        ],
    )
    def _encode(px_hbm, py_hbm, pz_hbm, latf_hbm, out_hbm,
                spm, bounce, pxb, pyb, pzb, idxb, wb, rowsb, outb, sem):
        wid = lax.axis_index("s") * 2 + lax.axis_index("c")
        sid = lax.axis_index("s")
        iota = lax.iota(jnp.int32, 16)
        zf = jnp.zeros((16,), jnp.float32)
        del iota

        def stage(src_base, dst_base, nf_padded):
            def sbody(i, c2):
                o = i * Q + sid * BNC
                pltpu.sync_copy(
                    latf_hbm.at[pl.ds(src_base + o, BNC)], bounce)
                pltpu.sync_copy(bounce, spm.at[pl.ds(dst_base + o, BNC)])
                return c2
            lax.fori_loop(0, nf_padded // Q, sbody, 0)

        for lvls, src in PASSES:
            k = len(lvls)
            nflt = k * 8 * 2 * C
            ndma = nflt // GB
            hashed = METH[lvls[0]] == "hash"

            plsc.subcore_barrier()
            if src == "spm":
                if hashed:
                    off_a, fpad, _rem = HSTG[lvls[0]]
                    stage(2 * off_a, 0, fpad)
                else:
                    for sl in lvls:
                        start, fpad, woff = STG[sl]
                        stage(2 * start, woff, fpad)
                plsc.subcore_barrier()

            gather_src = spm if src == "spm" else latf_hbm

            def chunk_body(t, carry, lvls=lvls, k=k, ndma=ndma,
                           src=src, gather_src=gather_src):
                base = wid * PW + t * C
                pltpu.sync_copy(px_hbm.at[pl.ds(base, C)], pxb)
                pltpu.sync_copy(py_hbm.at[pl.ds(base, C)], pyb)
                pltpu.sync_copy(pz_hbm.at[pl.ds(base, C)], pzb)

                def compute_group(g, c2):
                    g16 = g * 16
                    x = pxb[pl.ds(g16, 16)]
                    y = pyb[pl.ds(g16, 16)]
                    z = pzb[pl.ds(g16, 16)]
                    for li, l in enumerate(lvls):
                        res = RES[l]
                        sx = x * jnp.float32(res)
                        sy = y * jnp.float32(res)
                        sz = z * jnp.float32(res)
                        ix = sx.astype(jnp.int32)
                        iy = sy.astype(jnp.int32)
                        iz = sz.astype(jnp.int32)
                        fx = sx - ix.astype(jnp.float32)
                        fy = sy - iy.astype(jnp.float32)
                        fz = sz - iz.astype(jnp.float32)
                        gx = 1.0 - fx
                        gy = 1.0 - fy
                        gz = 1.0 - fz
                        wxy = (gx * gy, gx * fy, fx * gy, fx * fy)
                        wz = (gz, fz)
                        if METH[l] == "hash":
                            ts = (ix, ix + 1)
                            hy0 = iy * P1
                            us = (hy0, hy0 + P1)
                            hz0 = iz * P2
                            vs = (hz0, hz0 + P2)
                        elif src == "spm":
                            start, _size, woff = STG[l]
                            r2 = res * res
                            kc = 2 * OFF[l] + woff - 2 * start
                            t0 = (ix + res) * (2 * r2) + kc
                            ts = (t0, t0 + 2 * r2)
                            u0 = iy * (2 * res)
                            us = (u0, u0 + 2 * res)
                            izz = iz + iz
                            vs = (izz, izz + 2)
                        else:
                            r2 = res * res
                            t0 = (ix + res) * r2 + OFF[l]
                            ts = (t0, t0 + r2)
                            u0 = iy * res
                            us = (u0, u0 + res)
                            vs = (iz, iz + 1)
                        for dx in range(2):
                            for dy in range(2):
                                for dz in range(2):
                                    cc = dx * 4 + dy * 2 + dz
                                    if METH[l] == "hash":
                                        h2 = (ts[dx] ^ us[dy] ^ vs[dz]) & MASK
                                        e0 = h2 + h2 + (2 * HSTG[l][2])
                                    elif src == "spm":
                                        e0 = ts[dx] + us[dy] + vs[dz]
                                    else:
                                        idx = jnp.minimum(
                                            ts[dx] + us[dy] + vs[dz], RMAX)
                                        e0 = idx + idx
                                    lc = li * 8 + cc
                                    idxb[pl.ds(lc * 2 * C + g16, 16)] = e0
                                    idxb[pl.ds(lc * 2 * C + C + g16, 16)] = (
                                        e0 + 1)
                                    wb[pl.ds(lc * C + g16, 16)] = (
                                        wxy[dx * 2 + dy] * wz[dz])
                    return c2

                lax.fori_loop(0, NG, compute_group, 0)

                def fire(j, c2):
                    pltpu.make_async_copy(
                        gather_src.at[idxb.at[pl.ds(j * GB, GB)]],
                        rowsb.at[pl.ds(j * GB, GB)],
                        sem,
                    ).start()
                    return c2

                lax.fori_loop(0, ndma, fire, 0)

                def drain(j, c2):
                    pltpu.make_async_copy(
                        gather_src.at[idxb.at[pl.ds(j * GB, GB)]],
                        rowsb.at[pl.ds(j * GB, GB)],
                        sem,
                    ).wait()
                    return c2

                lax.fori_loop(0, ndma, drain, 0)

                def accum_group(g, c2):
                    g16 = g * 16
                    for li in range(k):
                        a0 = zf
                        a1 = zf
                        for cc in range(8):
                            lc = li * 8 + cc
                            w = wb[pl.ds(lc * C + g16, 16)]
                            r0 = rowsb[pl.ds(lc * 2 * C + g16, 16)]
                            r1 = rowsb[pl.ds(lc * 2 * C + C + g16, 16)]
                            a0 = a0 + w * r0
                            a1 = a1 + w * r1
                        outb[pl.ds((2 * li) * C + g16, 16)] = a0
                        outb[pl.ds((2 * li + 1) * C + g16, 16)] = a1
                    return c2

                lax.fori_loop(0, NG, accum_group, 0)

                q = wid * NCH + t
                pltpu.sync_copy(
                    outb.at[pl.ds(0, k * 2 * C)],
                    out_hbm.at[pl.ds(
                        q * 2 * LVLS * C + 2 * lvls[0] * C, k * 2 * C)])
                return carry

            lax.fori_loop(0, NCH, chunk_body, 0)

    return _encode


def kernel(pos, latents):
    px = pos[:, 0]
    py = pos[:, 1]
    pz = pos[:, 2]
    latf = jnp.reshape(latents, (-1,))
    latf = jnp.pad(latf, (0, LATF_PAD - latf.shape[0]))
    flat = _build_encoder()(px, py, pz, latf)
    cube = jnp.reshape(flat, (NPTS // C, 2 * LVLS, C))
    return jnp.reshape(jnp.transpose(cube, (0, 2, 1)), (NPTS, 2 * LVLS))
